# baseline (speedup 1.0000x reference)
import functools

import jax
import jax.numpy as jnp
from jax import lax
from jax.experimental import pallas as pl
from jax.experimental.pallas import tpu as pltpu
from jax.experimental.pallas import tpu_sc as plsc

VOCAB = 1000000
EMBED = 64
MAXLEN = 512
BATCH = 4096
SEQ = 200

NC = 2
NS = 16
NW = NC * NS

ROWS = BATCH * SEQ
R_PER_W = ROWS // NW
CHUNK = 1024
G_SUB = CHUNK // 128
N_CHUNKS = R_PER_W // CHUNK
LANES = 16
VPR = EMBED // LANES


def _body(x_hbm, tok_hbm, pos_hbm, out_hbm, idx_v, rows_v, pos_v, sem):
    wid = lax.axis_index("s") * NC + lax.axis_index("c")
    base = wid * R_PER_W

    pltpu.sync_copy(pos_hbm.at[pl.ds(0, SEQ)], pos_v)

    def chunk_body(g, carry):
        off = pl.multiple_of(base + g * CHUNK, CHUNK)
        irow = pl.multiple_of(off // 128, G_SUB)

        pltpu.sync_copy(x_hbm.at[pl.ds(irow, G_SUB)], idx_v)

        descs = []
        for j in range(G_SUB):
            descs.append(
                pltpu.async_copy(
                    tok_hbm.at[idx_v.at[j]],
                    rows_v.at[pl.ds(j * 128, 128)],
                    sem,
                )
            )
        for d in descs:
            d.wait()

        p0 = lax.rem(off, SEQ)

        def row_body(t, p):
            r0 = 2 * t
            p1 = p + 1
            p1 = jnp.where(p1 >= SEQ, p1 - SEQ, p1)
            for k in range(VPR):
                sl = pl.ds(k * LANES, LANES)
                plsc.addupdate(rows_v.at[r0, sl], pos_v[p, sl])
            for k in range(VPR):
                sl = pl.ds(k * LANES, LANES)
                plsc.addupdate(rows_v.at[r0 + 1, sl], pos_v[p1, sl])
            p2 = p1 + 1
            p2 = jnp.where(p2 >= SEQ, p2 - SEQ, p2)
            return p2

        lax.fori_loop(0, CHUNK // 2, row_body, p0)

        pltpu.sync_copy(rows_v, out_hbm.at[pl.ds(off, CHUNK)])
        return carry

    lax.fori_loop(0, N_CHUNKS, chunk_body, 0)


@jax.jit
def kernel(x, token_table, pos_table):
    x2 = x.reshape(ROWS // 128, 128)
    mesh = plsc.VectorSubcoreMesh(core_axis_name="c", subcore_axis_name="s")
    run = functools.partial(
        pl.kernel,
        mesh=mesh,
        out_type=jax.ShapeDtypeStruct((ROWS, EMBED), jnp.float32),
        scratch_types=[
            pltpu.VMEM((G_SUB, 128), jnp.int32),
            pltpu.VMEM((CHUNK, EMBED), jnp.float32),
            pltpu.VMEM((SEQ, EMBED), jnp.float32),
            pltpu.SemaphoreType.DMA,
        ],
        compiler_params=pltpu.CompilerParams(use_tc_tiling_on_sc=False),
    )(_body)
    out = run(x2, token_table, pos_table)
    return out.reshape(BATCH, SEQ, EMBED)


# speedup vs baseline: 1.0787x; 1.0787x over previous
import functools

import jax
import jax.numpy as jnp
from jax import lax
from jax.experimental import pallas as pl
from jax.experimental.pallas import tpu as pltpu
from jax.experimental.pallas import tpu_sc as plsc

VOCAB = 1000000
EMBED = 64
MAXLEN = 512
BATCH = 4096
SEQ = 200

NC = 2
NS = 16
NW = NC * NS

ROWS = BATCH * SEQ
R_PER_W = ROWS // NW
CHUNK = 512
G_SUB = CHUNK // 128
N_CHUNKS = R_PER_W // CHUNK
IDX_ROWS = R_PER_W // 128
LANES = 16
VPR = EMBED // LANES
RU = 4


def _body(x_hbm, tok_hbm, pos_hbm, out_hbm,
          idx_v, rows0, rows1, pos_v, sg0, sg1, sw0, sw1):
    rows = (rows0, rows1)
    sg = (sg0, sg1)
    sw = (sw0, sw1)
    wid = lax.axis_index("s") * NC + lax.axis_index("c")
    base = wid * R_PER_W

    pltpu.sync_copy(x_hbm.at[pl.ds(wid * IDX_ROWS, IDX_ROWS)], idx_v)
    pltpu.sync_copy(pos_hbm.at[pl.ds(0, SEQ)], pos_v)

    def fire_gathers(c, b):
        for j in range(G_SUB):
            pltpu.async_copy(
                tok_hbm.at[idx_v.at[c * G_SUB + j]],
                rows[b].at[pl.ds(j * 128, 128)],
                sg[b],
            )

    def drain_gathers(c, b):
        for j in range(G_SUB):
            pltpu.make_async_copy(
                tok_hbm.at[idx_v.at[c * G_SUB + j]],
                rows[b].at[pl.ds(j * 128, 128)],
                sg[b],
            ).wait()

    def add_pos(b, off):
        p0 = lax.rem(off, SEQ)

        def row4(t, p):
            r = RU * t
            for u in range(RU):
                pu = p + u
                pu = jnp.where(pu >= SEQ, pu - SEQ, pu)
                for k in range(VPR):
                    sl = pl.ds(k * LANES, LANES)
                    plsc.addupdate(rows[b].at[r + u, sl], pos_v[pu, sl])
            p4 = p + RU
            p4 = jnp.where(p4 >= SEQ, p4 - SEQ, p4)
            return p4

        lax.fori_loop(0, CHUNK // RU, row4, p0)

    fire_gathers(0, 0)

    def pair_body(i, carry):
        G = 2 * i
        for b in range(2):
            c = G + b
            off = pl.multiple_of(base + c * CHUNK, CHUNK)
            nb = 1 - b

            @pl.when(c + 1 < N_CHUNKS)
            def _():
                @pl.when(c >= 1)
                def _():
                    off_prev = pl.multiple_of(base + (c - 1) * CHUNK, CHUNK)
                    pltpu.make_async_copy(
                        rows[nb], out_hbm.at[pl.ds(off_prev, CHUNK)], sw[nb]
                    ).wait()

                fire_gathers(c + 1, nb)

            drain_gathers(c, b)
            add_pos(b, off)
            pltpu.async_copy(rows[b], out_hbm.at[pl.ds(off, CHUNK)], sw[b])
        return carry

    lax.fori_loop(0, N_CHUNKS // 2, pair_body, 0)

    for c in (N_CHUNKS - 2, N_CHUNKS - 1):
        b = c % 2
        off = pl.multiple_of(base + c * CHUNK, CHUNK)
        pltpu.make_async_copy(
            rows[b], out_hbm.at[pl.ds(off, CHUNK)], sw[b]
        ).wait()


def kernel(x, token_table, pos_table):
    x2 = x.reshape(ROWS // 128, 128)
    mesh = plsc.VectorSubcoreMesh(core_axis_name="c", subcore_axis_name="s")
    run = functools.partial(
        pl.kernel,
        mesh=mesh,
        out_type=jax.ShapeDtypeStruct((ROWS, EMBED), jnp.float32),
        scratch_types=[
            pltpu.VMEM((IDX_ROWS, 128), jnp.int32),
            pltpu.VMEM((CHUNK, EMBED), jnp.float32),
            pltpu.VMEM((CHUNK, EMBED), jnp.float32),
            pltpu.VMEM((SEQ, EMBED), jnp.float32),
            pltpu.SemaphoreType.DMA,
            pltpu.SemaphoreType.DMA,
            pltpu.SemaphoreType.DMA,
            pltpu.SemaphoreType.DMA,
        ],
        compiler_params=pltpu.CompilerParams(use_tc_tiling_on_sc=False),
    )(_body)
    out = run(x2, token_table, pos_table)
    return out.reshape(BATCH, SEQ, EMBED)
